# Initial kernel scaffold; baseline (speedup 1.0000x reference)
import jax
import jax.numpy as jnp
from jax.experimental import pallas as pl


def kernel(category, noise, edge_index, W0, b0, W1, b1, W2, b2):
    raise NotImplementedError("write your pallas kernel here")


# speedup vs baseline: 6.1876x; 6.1876x over previous
import functools

import jax
import jax.numpy as jnp
import numpy as np
from jax import lax
from jax.experimental import pallas as pl
from jax.experimental.pallas import tpu as pltpu
from jax.experimental.pallas import tpu_sc as plsc

N = 10000
D = 128
E = 320000
NC, NS = 2, 16
NH = 5000
STR = 6000
ACC = 5120
TWB = ACC // NS
GPAD = 20480
C = 128
ET = 20480
ECH = ET // C
EPAD = NS * ET
QCH = 40
MAGIC = np.int32(0x5F3759DF)

_mesh = plsc.VectorSubcoreMesh(core_axis_name="c", subcore_axis_name="s")

_CHUNKS = ((0, 128), (128, 128), (256, 64))


def _fill16(ref, n, val, dtype):
    v = jnp.full((16,), val, dtype)

    def body(i, carry):
        ref[i, pl.ds(0, 16)] = v
        return carry

    lax.fori_loop(0, n, body, 0)


def _fill_gbuf(gbuf, val):
    z = jnp.full((16,), val, jnp.float32)

    def body(i, carry):
        for cc in range(8):
            gbuf[i, pl.ds(cc * 16, 16)] = z
        return carry

    lax.fori_loop(0, 128, body, 0)


def _zero_gbuf(gbuf):
    _fill_gbuf(gbuf, 0.0)


def _stage_quarter(src_hbm, dst_hbm, src_v, dst_v, s, c, qq):
    base = c * NH
    pltpu.sync_copy(src_hbm.at[s, pl.ds(qq * QCH, QCH)], src_v)
    pltpu.sync_copy(dst_hbm.at[s, pl.ds(qq * QCH, QCH)], dst_v)

    def body(j, carry):
        for cc in range(8):
            sl = pl.ds(cc * 16, 16)
            v = src_v[j, sl]
            src_v[j, sl] = jnp.where(v >= NH, v + (STR - NH), v)
            w = dst_v[j, sl]
            local = w - base
            ok = (local >= 0) & (local < NH)
            dst_v[j, sl] = jnp.where(ok, local, NH + (w & 63))
        return carry

    lax.fori_loop(0, QCH, body, 0)


def _newton_rsqrt(d16):
    half = d16 * 0.5
    bits = lax.bitcast_convert_type(d16, jnp.int32)
    i = MAGIC - lax.shift_right_logical(bits, 1)
    y = lax.bitcast_convert_type(i, jnp.float32)
    for _ in range(3):
        y = y * (1.5 - half * y * y)
    return jnp.where(d16 > 0.5, y, 0.0)


def _sc_prep_body(xg_hbm, src_hbm, dst_hbm, gout, dinv_out,
                  src_v, dst_v, buf1, dv, ibuf, dacc, sem0):
    c = lax.axis_index("c")
    s = lax.axis_index("s")

    _zero_gbuf(buf1)
    pltpu.sync_copy(buf1, dacc.at[pl.ds(s * TWB, 128)])
    pltpu.sync_copy(buf1, dacc.at[pl.ds(s * TWB + 128, 128)])
    pltpu.sync_copy(buf1.at[pl.ds(0, 64)], dacc.at[pl.ds(s * TWB + 256, 64)])
    _fill_gbuf(buf1, 1.0)
    plsc.subcore_barrier()

    def douter(qq, carry):
        _stage_quarter(src_hbm, dst_hbm, src_v, dst_v, s, c, qq)

        def dbody(j, inner):
            pltpu.sync_copy(buf1, dacc.at[dst_v.at[j]], add=True)
            return inner

        lax.fori_loop(0, QCH, dbody, 0)
        return carry

    lax.fori_loop(0, ECH // QCH, douter, 0)
    plsc.subcore_barrier()

    for off, cnt in _CHUNKS:
        gslice = buf1.at[pl.ds(0, cnt)] if cnt != 128 else buf1
        pltpu.sync_copy(dacc.at[pl.ds(s * TWB + off, cnt)], gslice)

        def ibody(i, carry, off=off):
            d16 = buf1[i, pl.ds(0, 16)]
            dv[off + i, pl.ds(0, 16)] = _newton_rsqrt(d16)
            return carry

        lax.fori_loop(0, cnt, ibody, 0)
    pltpu.sync_copy(dv, dinv_out.at[c, pl.ds(s * TWB, TWB)])

    for off, cnt in _CHUNKS:
        gslice = buf1.at[pl.ds(0, cnt)] if cnt != 128 else buf1
        nrow = c * NH + s * TWB + off

        def xbody(j, carry, nrow=nrow):
            ibuf[pl.ds(j * 16, 16)] = nrow + j * 16 + lax.iota(jnp.int32, 16)
            return carry

        lax.fori_loop(0, cnt // 16, xbody, 0)
        islice = ibuf.at[pl.ds(0, cnt)] if cnt != 128 else ibuf
        pltpu.async_copy(xg_hbm.at[islice], gslice, sem0).wait()

        def gbody(i, carry, off=off):
            d = dv[off + i, pl.ds(0, 16)][0]
            for cc in range(8):
                sl = pl.ds(cc * 16, 16)
                buf1[i, sl] = buf1[i, sl] * d
            return carry

        lax.fori_loop(0, cnt, gbody, 0)
        rows = c * STR + s * TWB + off
        pltpu.sync_copy(gslice, gout.at[pl.ds(rows, cnt)])


def _sc_hop_body(gtab, src_hbm, dst_hbm, dinv_in, gout,
                 src_v, dst_v, buf0, buf1, dv, accum, sem0, sem1):
    c = lax.axis_index("c")
    s = lax.axis_index("s")
    pltpu.sync_copy(dinv_in.at[c, pl.ds(s * TWB, TWB)], dv)

    _zero_gbuf(buf1)
    pltpu.sync_copy(buf1, accum.at[pl.ds(s * TWB, 128)])
    pltpu.sync_copy(buf1, accum.at[pl.ds(s * TWB + 128, 128)])
    pltpu.sync_copy(buf1.at[pl.ds(0, 64)], accum.at[pl.ds(s * TWB + 256, 64)])
    plsc.subcore_barrier()

    def outer(qq, carry):
        _stage_quarter(src_hbm, dst_hbm, src_v, dst_v, s, c, qq)

        def body(it, inner):
            ch0 = it * 2
            ch1 = it * 2 + 1
            d0 = pltpu.async_copy(gtab.at[src_v.at[ch0]], buf0, sem0)
            d1 = pltpu.async_copy(gtab.at[src_v.at[ch1]], buf1, sem1)
            d0.wait()
            pltpu.sync_copy(buf0, accum.at[dst_v.at[ch0]], add=True)
            d1.wait()
            pltpu.sync_copy(buf1, accum.at[dst_v.at[ch1]], add=True)
            return inner

        lax.fori_loop(0, QCH // 2, body, 0)
        return carry

    lax.fori_loop(0, ECH // QCH, outer, 0)
    plsc.subcore_barrier()

    for off, cnt in _CHUNKS:
        gslice = buf1.at[pl.ds(0, cnt)] if cnt != 128 else buf1
        pltpu.sync_copy(accum.at[pl.ds(s * TWB + off, cnt)], gslice)

        def sbody(i, carry, off=off):
            d = dv[off + i, pl.ds(0, 16)][0]
            d2 = d * d
            for cc in range(8):
                sl = pl.ds(cc * 16, 16)
                buf1[i, sl] = buf1[i, sl] * d2
            return carry

        lax.fori_loop(0, cnt, sbody, 0)
        rows = c * STR + s * TWB + off
        pltpu.sync_copy(gslice, gout.at[pl.ds(rows, cnt)])


_sc_prep = functools.partial(
    pl.kernel,
    out_type=(
        jax.ShapeDtypeStruct((GPAD, D), jnp.float32),
        jax.ShapeDtypeStruct((NC, ACC, 16), jnp.float32),
    ),
    mesh=_mesh,
    scratch_types=[
        pltpu.VMEM((QCH, C), jnp.int32),
        pltpu.VMEM((QCH, C), jnp.int32),
        pltpu.VMEM((C, D), jnp.float32),
        pltpu.VMEM((TWB, 16), jnp.float32),
        pltpu.VMEM((C,), jnp.int32),
        pltpu.VMEM_SHARED((ACC, D), jnp.float32),
        pltpu.SemaphoreType.DMA,
    ],
)(_sc_prep_body)

_sc_hop = functools.partial(
    pl.kernel,
    out_type=jax.ShapeDtypeStruct((GPAD, D), jnp.float32),
    mesh=_mesh,
    scratch_types=[
        pltpu.VMEM((QCH, C), jnp.int32),
        pltpu.VMEM((QCH, C), jnp.int32),
        pltpu.VMEM((C, D), jnp.float32),
        pltpu.VMEM((C, D), jnp.float32),
        pltpu.VMEM((TWB, 16), jnp.float32),
        pltpu.VMEM_SHARED((ACC, D), jnp.float32),
        pltpu.SemaphoreType.DMA,
        pltpu.SemaphoreType.DMA,
    ],
)(_sc_hop_body)


_R = 1000


def _g_idx(i):
    return (i // 5) * (STR // _R) + (i % 5)


def _tc_layer_body(g1_ref, g2_ref, g3_ref, d_ref, x_ref, w_ref, b_ref, *out):
    dv = d_ref[0, :, :1]
    sq = jnp.where(dv > 0, 1.0 / dv, 0.0)
    h1 = g1_ref[...] * sq
    h2 = g2_ref[...] * sq
    h3 = g3_ref[...] * sq
    big = jnp.concatenate([x_ref[...], h1, h2, h3], axis=1)
    o = jnp.dot(big, w_ref[...], preferred_element_type=jnp.float32) + b_ref[...]
    xn = jnp.where(o > 0, o, 0.25 * o)
    out[0][...] = xn
    if len(out) > 1:
        out[1][...] = xn * dv


def _tc_layer(g1, g2, g3, dinv, x, wstack, b, last):
    out_shape = [jax.ShapeDtypeStruct((N, D), jnp.float32)]
    out_specs = [pl.BlockSpec((_R, D), lambda i: (i, 0))]
    if not last:
        out_shape.append(jax.ShapeDtypeStruct((GPAD, D), jnp.float32))
        out_specs.append(pl.BlockSpec((_R, D), lambda i: (_g_idx(i), 0)))
    g_spec = pl.BlockSpec((_R, D), lambda i: (_g_idx(i), 0))
    res = pl.pallas_call(
        _tc_layer_body,
        grid=(N // _R,),
        in_specs=[
            g_spec,
            g_spec,
            g_spec,
            pl.BlockSpec((1, _R, 16), lambda i: (i // 5, i % 5, 0)),
            pl.BlockSpec((_R, D), lambda i: (i, 0)),
            pl.BlockSpec((4 * D, D), lambda i: (0, 0)),
            pl.BlockSpec((1, D), lambda i: (0, 0)),
        ],
        out_specs=out_specs if not last else out_specs[0],
        out_shape=out_shape if not last else out_shape[0],
    )(g1, g2, g3, dinv, x, wstack, b)
    return res


def kernel(category, noise, edge_index, W0, b0, W1, b1, W2, b2):
    src = edge_index[0]
    dst = edge_index[1]
    pad = EPAD - E
    ar = jnp.arange(pad, dtype=jnp.int32)
    src_p = jnp.concatenate([src, ar % N]).reshape(NS, ECH, C)
    dst_p = jnp.concatenate([dst, jnp.full((pad,), N, jnp.int32)]).reshape(NS, ECH, C)

    x0 = jnp.concatenate([category, noise], axis=1)
    xpad = jnp.pad(x0, ((0, GPAD - N), (0, 0)))

    g0, dinv = _sc_prep(xpad, src_p, dst_p)
    x = x0
    for l, (W, b) in enumerate(((W0, b0), (W1, b1), (W2, b2))):
        g1 = _sc_hop(g0, src_p, dst_p, dinv)
        g2 = _sc_hop(g1, src_p, dst_p, dinv)
        g3 = _sc_hop(g2, src_p, dst_p, dinv)
        last = l == 2
        res = _tc_layer(g1, g2, g3, dinv, x, W.reshape(4 * D, D),
                        b.reshape(1, D), last)
        if last:
            return res
        x, g0 = res


# speedup vs baseline: 6.2156x; 1.0045x over previous
import functools

import jax
import jax.numpy as jnp
import numpy as np
from jax import lax
from jax.experimental import pallas as pl
from jax.experimental.pallas import tpu as pltpu
from jax.experimental.pallas import tpu_sc as plsc

N = 10000
D = 128
E = 320000
NC, NS = 2, 16
NH = 5000
STR = 6000
ACC = 5120
TWB = ACC // NS
GPAD = 20480
C = 128
ET = 20480
ECH = ET // C
EPAD = NS * ET
QCH = 40
MAGIC = np.int32(0x5F3759DF)

_mesh = plsc.VectorSubcoreMesh(core_axis_name="c", subcore_axis_name="s")

_CHUNKS = ((0, 128), (128, 128), (256, 64))


def _fill16(ref, n, val, dtype):
    v = jnp.full((16,), val, dtype)

    def body(i, carry):
        ref[i, pl.ds(0, 16)] = v
        return carry

    lax.fori_loop(0, n, body, 0)


def _fill_gbuf(gbuf, val):
    z = jnp.full((16,), val, jnp.float32)

    def body(i, carry):
        for cc in range(8):
            gbuf[i, pl.ds(cc * 16, 16)] = z
        return carry

    lax.fori_loop(0, 128, body, 0)


def _zero_gbuf(gbuf):
    _fill_gbuf(gbuf, 0.0)


def _stage_quarter(src_hbm, dst_hbm, src_v, dst_v, s, c, qq):
    base = c * NH
    pltpu.sync_copy(src_hbm.at[s, pl.ds(qq * QCH, QCH)], src_v)
    pltpu.sync_copy(dst_hbm.at[s, pl.ds(qq * QCH, QCH)], dst_v)

    def body(j, carry):
        for cc in range(8):
            sl = pl.ds(cc * 16, 16)
            v = src_v[j, sl]
            src_v[j, sl] = jnp.where(v >= NH, v + (STR - NH), v)
            w = dst_v[j, sl]
            local = w - base
            ok = (local >= 0) & (local < NH)
            dst_v[j, sl] = jnp.where(ok, local, NH + (w & 63))
        return carry

    lax.fori_loop(0, QCH, body, 0)


def _newton_rsqrt(d16):
    half = d16 * 0.5
    bits = lax.bitcast_convert_type(d16, jnp.int32)
    i = MAGIC - lax.shift_right_logical(bits, 1)
    y = lax.bitcast_convert_type(i, jnp.float32)
    for _ in range(3):
        y = y * (1.5 - half * y * y)
    return jnp.where(d16 > 0.5, y, 0.0)


def _sc_prep_body(xg_hbm, src_hbm, dst_hbm, gout, dinv_out, srcT, dstT,
                  src_v, dst_v, buf1, dv, ibuf, dacc, sem0):
    c = lax.axis_index("c")
    s = lax.axis_index("s")

    _zero_gbuf(buf1)
    pltpu.sync_copy(buf1, dacc.at[pl.ds(s * TWB, 128)])
    pltpu.sync_copy(buf1, dacc.at[pl.ds(s * TWB + 128, 128)])
    pltpu.sync_copy(buf1.at[pl.ds(0, 64)], dacc.at[pl.ds(s * TWB + 256, 64)])
    _fill_gbuf(buf1, 1.0)
    plsc.subcore_barrier()

    def douter(qq, carry):
        _stage_quarter(src_hbm, dst_hbm, src_v, dst_v, s, c, qq)
        pltpu.sync_copy(src_v, srcT.at[c, s, pl.ds(qq * QCH, QCH)])
        pltpu.sync_copy(dst_v, dstT.at[c, s, pl.ds(qq * QCH, QCH)])

        def dbody(j, inner):
            pltpu.sync_copy(buf1, dacc.at[dst_v.at[j]], add=True)
            return inner

        lax.fori_loop(0, QCH, dbody, 0)
        return carry

    lax.fori_loop(0, ECH // QCH, douter, 0)
    plsc.subcore_barrier()

    for off, cnt in _CHUNKS:
        gslice = buf1.at[pl.ds(0, cnt)] if cnt != 128 else buf1
        pltpu.sync_copy(dacc.at[pl.ds(s * TWB + off, cnt)], gslice)

        def ibody(i, carry, off=off):
            d16 = buf1[i, pl.ds(0, 16)]
            dv[off + i, pl.ds(0, 16)] = _newton_rsqrt(d16)
            return carry

        lax.fori_loop(0, cnt, ibody, 0)
    pltpu.sync_copy(dv, dinv_out.at[c, pl.ds(s * TWB, TWB)])

    for off, cnt in _CHUNKS:
        gslice = buf1.at[pl.ds(0, cnt)] if cnt != 128 else buf1
        nrow = c * NH + s * TWB + off

        def xbody(j, carry, nrow=nrow):
            ibuf[pl.ds(j * 16, 16)] = nrow + j * 16 + lax.iota(jnp.int32, 16)
            return carry

        lax.fori_loop(0, cnt // 16, xbody, 0)
        islice = ibuf.at[pl.ds(0, cnt)] if cnt != 128 else ibuf
        pltpu.async_copy(xg_hbm.at[islice], gslice, sem0).wait()

        def gbody(i, carry, off=off):
            d = dv[off + i, pl.ds(0, 16)][0]
            for cc in range(8):
                sl = pl.ds(cc * 16, 16)
                buf1[i, sl] = buf1[i, sl] * d
            return carry

        lax.fori_loop(0, cnt, gbody, 0)
        rows = c * STR + s * TWB + off
        pltpu.sync_copy(gslice, gout.at[pl.ds(rows, cnt)])


def _sc_hop_body(gtab, srcT, dstT, dinv_in, gout,
                 src_v, dst_v, buf0, buf1, dv, accum, sem0, sem1):
    c = lax.axis_index("c")
    s = lax.axis_index("s")
    pltpu.sync_copy(dinv_in.at[c, pl.ds(s * TWB, TWB)], dv)

    _zero_gbuf(buf1)
    pltpu.sync_copy(buf1, accum.at[pl.ds(s * TWB, 128)])
    pltpu.sync_copy(buf1, accum.at[pl.ds(s * TWB + 128, 128)])
    pltpu.sync_copy(buf1.at[pl.ds(0, 64)], accum.at[pl.ds(s * TWB + 256, 64)])
    plsc.subcore_barrier()

    def outer(qq, carry):
        pltpu.sync_copy(srcT.at[c, s, pl.ds(qq * QCH, QCH)], src_v)
        pltpu.sync_copy(dstT.at[c, s, pl.ds(qq * QCH, QCH)], dst_v)

        def body(it, inner):
            ch0 = it * 2
            ch1 = it * 2 + 1
            d0 = pltpu.async_copy(gtab.at[src_v.at[ch0]], buf0, sem0)
            d1 = pltpu.async_copy(gtab.at[src_v.at[ch1]], buf1, sem1)
            d0.wait()
            pltpu.sync_copy(buf0, accum.at[dst_v.at[ch0]], add=True)
            d1.wait()
            pltpu.sync_copy(buf1, accum.at[dst_v.at[ch1]], add=True)
            return inner

        lax.fori_loop(0, QCH // 2, body, 0)
        return carry

    lax.fori_loop(0, ECH // QCH, outer, 0)
    plsc.subcore_barrier()

    for off, cnt in _CHUNKS:
        gslice = buf1.at[pl.ds(0, cnt)] if cnt != 128 else buf1
        pltpu.sync_copy(accum.at[pl.ds(s * TWB + off, cnt)], gslice)

        def sbody(i, carry, off=off):
            d = dv[off + i, pl.ds(0, 16)][0]
            d2 = d * d
            for cc in range(8):
                sl = pl.ds(cc * 16, 16)
                buf1[i, sl] = buf1[i, sl] * d2
            return carry

        lax.fori_loop(0, cnt, sbody, 0)
        rows = c * STR + s * TWB + off
        pltpu.sync_copy(gslice, gout.at[pl.ds(rows, cnt)])


_sc_prep = functools.partial(
    pl.kernel,
    out_type=(
        jax.ShapeDtypeStruct((GPAD, D), jnp.float32),
        jax.ShapeDtypeStruct((NC, ACC, 16), jnp.float32),
        jax.ShapeDtypeStruct((NC, NS, ECH, C), jnp.int32),
        jax.ShapeDtypeStruct((NC, NS, ECH, C), jnp.int32),
    ),
    mesh=_mesh,
    scratch_types=[
        pltpu.VMEM((QCH, C), jnp.int32),
        pltpu.VMEM((QCH, C), jnp.int32),
        pltpu.VMEM((C, D), jnp.float32),
        pltpu.VMEM((TWB, 16), jnp.float32),
        pltpu.VMEM((C,), jnp.int32),
        pltpu.VMEM_SHARED((ACC, D), jnp.float32),
        pltpu.SemaphoreType.DMA,
    ],
)(lambda xg, srch, dsth, gout, dinv_out, srcT, dstT, *scr: _sc_prep_body(
    xg, srch, dsth, gout, dinv_out, srcT, dstT, *scr))

_sc_hop = functools.partial(
    pl.kernel,
    out_type=jax.ShapeDtypeStruct((GPAD, D), jnp.float32),
    mesh=_mesh,
    scratch_types=[
        pltpu.VMEM((QCH, C), jnp.int32),
        pltpu.VMEM((QCH, C), jnp.int32),
        pltpu.VMEM((C, D), jnp.float32),
        pltpu.VMEM((C, D), jnp.float32),
        pltpu.VMEM((TWB, 16), jnp.float32),
        pltpu.VMEM_SHARED((ACC, D), jnp.float32),
        pltpu.SemaphoreType.DMA,
        pltpu.SemaphoreType.DMA,
    ],
)(_sc_hop_body)


_R = 1000


def _g_idx(i):
    return (i // 5) * (STR // _R) + (i % 5)


def _tc_layer_body(g1_ref, g2_ref, g3_ref, d_ref, x_ref, w_ref, b_ref, *out):
    dv = d_ref[0, :, :1]
    sq = jnp.where(dv > 0, 1.0 / dv, 0.0)
    h1 = g1_ref[...] * sq
    h2 = g2_ref[...] * sq
    h3 = g3_ref[...] * sq
    big = jnp.concatenate([x_ref[...], h1, h2, h3], axis=1)
    o = jnp.dot(big, w_ref[...], preferred_element_type=jnp.float32) + b_ref[...]
    xn = jnp.where(o > 0, o, 0.25 * o)
    out[0][...] = xn
    if len(out) > 1:
        out[1][...] = xn * dv


def _tc_layer(g1, g2, g3, dinv, x, wstack, b, last):
    out_shape = [jax.ShapeDtypeStruct((N, D), jnp.float32)]
    out_specs = [pl.BlockSpec((_R, D), lambda i: (i, 0))]
    if not last:
        out_shape.append(jax.ShapeDtypeStruct((GPAD, D), jnp.float32))
        out_specs.append(pl.BlockSpec((_R, D), lambda i: (_g_idx(i), 0)))
    g_spec = pl.BlockSpec((_R, D), lambda i: (_g_idx(i), 0))
    res = pl.pallas_call(
        _tc_layer_body,
        grid=(N // _R,),
        in_specs=[
            g_spec,
            g_spec,
            g_spec,
            pl.BlockSpec((1, _R, 16), lambda i: (i // 5, i % 5, 0)),
            pl.BlockSpec((_R, D), lambda i: (i, 0)),
            pl.BlockSpec((4 * D, D), lambda i: (0, 0)),
            pl.BlockSpec((1, D), lambda i: (0, 0)),
        ],
        out_specs=out_specs if not last else out_specs[0],
        out_shape=out_shape if not last else out_shape[0],
    )(g1, g2, g3, dinv, x, wstack, b)
    return res


def kernel(category, noise, edge_index, W0, b0, W1, b1, W2, b2):
    src = edge_index[0]
    dst = edge_index[1]
    pad = EPAD - E
    ar = jnp.arange(pad, dtype=jnp.int32)
    src_p = jnp.concatenate([src, ar % N]).reshape(NS, ECH, C)
    dst_p = jnp.concatenate([dst, jnp.full((pad,), N, jnp.int32)]).reshape(NS, ECH, C)

    x0 = jnp.concatenate([category, noise], axis=1)
    xpad = jnp.pad(x0, ((0, GPAD - N), (0, 0)))

    g0, dinv, srcT, dstT = _sc_prep(xpad, src_p, dst_p)
    x = x0
    for l, (W, b) in enumerate(((W0, b0), (W1, b1), (W2, b2))):
        g1 = _sc_hop(g0, srcT, dstT, dinv)
        g2 = _sc_hop(g1, srcT, dstT, dinv)
        g3 = _sc_hop(g2, srcT, dstT, dinv)
        last = l == 2
        res = _tc_layer(g1, g2, g3, dinv, x, W.reshape(4 * D, D),
                        b.reshape(1, D), last)
        if last:
            return res
        x, g0 = res


# speedup vs baseline: 6.3124x; 1.0156x over previous
import functools

import jax
import jax.numpy as jnp
import numpy as np
from jax import lax
from jax.experimental import pallas as pl
from jax.experimental.pallas import tpu as pltpu
from jax.experimental.pallas import tpu_sc as plsc

N = 10000
D = 128
E = 320000
NC, NS = 2, 16
NH = 5000
STR = 6000
ACC = 5120
TWB = ACC // NS
GPAD = 20480
C = 128
ET = 20480
ECH = ET // C
EPAD = NS * ET
QCH = 40
MAGIC = np.int32(0x5F3759DF)

_mesh = plsc.VectorSubcoreMesh(core_axis_name="c", subcore_axis_name="s")

_CHUNKS = ((0, 128), (128, 128), (256, 64))


def _fill16(ref, n, val, dtype):
    v = jnp.full((16,), val, dtype)

    def body(i, carry):
        ref[i, pl.ds(0, 16)] = v
        return carry

    lax.fori_loop(0, n, body, 0)


def _fill_gbuf(gbuf, val):
    z = jnp.full((16,), val, jnp.float32)

    def body(i, carry):
        for cc in range(8):
            gbuf[i, pl.ds(cc * 16, 16)] = z
        return carry

    lax.fori_loop(0, 128, body, 0)


def _zero_gbuf(gbuf):
    _fill_gbuf(gbuf, 0.0)


def _stage_quarter(src_hbm, dst_hbm, src_v, dst_v, s, c, qq):
    base = c * NH
    pltpu.sync_copy(src_hbm.at[s, pl.ds(qq * QCH, QCH)], src_v)
    pltpu.sync_copy(dst_hbm.at[s, pl.ds(qq * QCH, QCH)], dst_v)

    def body(j, carry):
        for cc in range(8):
            sl = pl.ds(cc * 16, 16)
            v = src_v[j, sl]
            src_v[j, sl] = jnp.where(v >= NH, v + (STR - NH), v)
            w = dst_v[j, sl]
            local = w - base
            ok = (local >= 0) & (local < NH)
            dst_v[j, sl] = jnp.where(ok, local, NH + (w & 63))
        return carry

    lax.fori_loop(0, QCH, body, 0)


def _newton_rsqrt(d16):
    half = d16 * 0.5
    bits = lax.bitcast_convert_type(d16, jnp.int32)
    i = MAGIC - lax.shift_right_logical(bits, 1)
    y = lax.bitcast_convert_type(i, jnp.float32)
    for _ in range(3):
        y = y * (1.5 - half * y * y)
    return jnp.where(d16 > 0.5, y, 0.0)


def _sc_prep_body(xg_hbm, src_hbm, dst_hbm, gout, dinv_out, srcT, dstT,
                  src_v, dst_v, buf1, dv, ibuf, dacc, sem0):
    c = lax.axis_index("c")
    s = lax.axis_index("s")

    _zero_gbuf(buf1)
    pltpu.sync_copy(buf1, dacc.at[pl.ds(s * TWB, 128)])
    pltpu.sync_copy(buf1, dacc.at[pl.ds(s * TWB + 128, 128)])
    pltpu.sync_copy(buf1.at[pl.ds(0, 64)], dacc.at[pl.ds(s * TWB + 256, 64)])
    _fill_gbuf(buf1, 1.0)
    plsc.subcore_barrier()

    def douter(qq, carry):
        _stage_quarter(src_hbm, dst_hbm, src_v, dst_v, s, c, qq)
        pltpu.sync_copy(src_v, srcT.at[c, s, pl.ds(qq * QCH, QCH)])
        pltpu.sync_copy(dst_v, dstT.at[c, s, pl.ds(qq * QCH, QCH)])

        def dbody(j, inner):
            pltpu.sync_copy(buf1, dacc.at[dst_v.at[j]], add=True)
            return inner

        lax.fori_loop(0, QCH, dbody, 0)
        return carry

    lax.fori_loop(0, ECH // QCH, douter, 0)
    plsc.subcore_barrier()

    for off, cnt in _CHUNKS:
        gslice = buf1.at[pl.ds(0, cnt)] if cnt != 128 else buf1
        pltpu.sync_copy(dacc.at[pl.ds(s * TWB + off, cnt)], gslice)

        def ibody(i, carry, off=off):
            d16 = buf1[i, pl.ds(0, 16)]
            dv[off + i, pl.ds(0, 16)] = _newton_rsqrt(d16)
            return carry

        lax.fori_loop(0, cnt, ibody, 0)
    pltpu.sync_copy(dv, dinv_out.at[c, pl.ds(s * TWB, TWB)])

    for off, cnt in _CHUNKS:
        gslice = buf1.at[pl.ds(0, cnt)] if cnt != 128 else buf1
        nrow = c * NH + s * TWB + off

        def xbody(j, carry, nrow=nrow):
            ibuf[pl.ds(j * 16, 16)] = nrow + j * 16 + lax.iota(jnp.int32, 16)
            return carry

        lax.fori_loop(0, cnt // 16, xbody, 0)
        islice = ibuf.at[pl.ds(0, cnt)] if cnt != 128 else ibuf
        pltpu.async_copy(xg_hbm.at[islice], gslice, sem0).wait()

        def gbody(i, carry, off=off):
            d = dv[off + i, pl.ds(0, 16)][0]
            for cc in range(8):
                sl = pl.ds(cc * 16, 16)
                buf1[i, sl] = buf1[i, sl] * d
            return carry

        lax.fori_loop(0, cnt, gbody, 0)
        rows = c * STR + s * TWB + off
        pltpu.sync_copy(gslice, gout.at[pl.ds(rows, cnt)])


def _sc_hop_body(gtab, srcT, dstT, dinv_in, gout,
                 src_v, dst_v, buf0, buf1, dv, accum, sem0, sem1, sem2, sem3):
    c = lax.axis_index("c")
    s = lax.axis_index("s")
    pltpu.sync_copy(dinv_in.at[c, pl.ds(s * TWB, TWB)], dv)

    _zero_gbuf(buf1)
    pltpu.sync_copy(buf1, accum.at[pl.ds(s * TWB, 128)])
    pltpu.sync_copy(buf1, accum.at[pl.ds(s * TWB + 128, 128)])
    pltpu.sync_copy(buf1.at[pl.ds(0, 64)], accum.at[pl.ds(s * TWB + 256, 64)])
    plsc.subcore_barrier()

    def outer(qq, carry):
        pltpu.sync_copy(srcT.at[c, s, pl.ds(qq * QCH, QCH)], src_v)
        pltpu.sync_copy(dstT.at[c, s, pl.ds(qq * QCH, QCH)], dst_v)

        def body(it, inner):
            ch0 = it * 2
            ch1 = it * 2 + 1
            d0 = pltpu.async_copy(gtab.at[src_v.at[ch0]], buf0, sem0)
            d1 = pltpu.async_copy(gtab.at[src_v.at[ch1]], buf1, sem1)
            d0.wait()
            s0 = pltpu.async_copy(buf0, accum.at[dst_v.at[ch0]], sem2, add=True)
            d1.wait()
            s1 = pltpu.async_copy(buf1, accum.at[dst_v.at[ch1]], sem3, add=True)
            s0.wait()
            s1.wait()
            return inner

        lax.fori_loop(0, QCH // 2, body, 0)
        return carry

    lax.fori_loop(0, ECH // QCH, outer, 0)
    plsc.subcore_barrier()

    for off, cnt in _CHUNKS:
        gslice = buf1.at[pl.ds(0, cnt)] if cnt != 128 else buf1
        pltpu.sync_copy(accum.at[pl.ds(s * TWB + off, cnt)], gslice)

        def sbody(i, carry, off=off):
            d = dv[off + i, pl.ds(0, 16)][0]
            d2 = d * d
            for cc in range(8):
                sl = pl.ds(cc * 16, 16)
                buf1[i, sl] = buf1[i, sl] * d2
            return carry

        lax.fori_loop(0, cnt, sbody, 0)
        rows = c * STR + s * TWB + off
        pltpu.sync_copy(gslice, gout.at[pl.ds(rows, cnt)])


_sc_prep = functools.partial(
    pl.kernel,
    out_type=(
        jax.ShapeDtypeStruct((GPAD, D), jnp.float32),
        jax.ShapeDtypeStruct((NC, ACC, 16), jnp.float32),
        jax.ShapeDtypeStruct((NC, NS, ECH, C), jnp.int32),
        jax.ShapeDtypeStruct((NC, NS, ECH, C), jnp.int32),
    ),
    mesh=_mesh,
    scratch_types=[
        pltpu.VMEM((QCH, C), jnp.int32),
        pltpu.VMEM((QCH, C), jnp.int32),
        pltpu.VMEM((C, D), jnp.float32),
        pltpu.VMEM((TWB, 16), jnp.float32),
        pltpu.VMEM((C,), jnp.int32),
        pltpu.VMEM_SHARED((ACC, D), jnp.float32),
        pltpu.SemaphoreType.DMA,
    ],
)(lambda xg, srch, dsth, gout, dinv_out, srcT, dstT, *scr: _sc_prep_body(
    xg, srch, dsth, gout, dinv_out, srcT, dstT, *scr))

_sc_hop = functools.partial(
    pl.kernel,
    out_type=jax.ShapeDtypeStruct((GPAD, D), jnp.float32),
    mesh=_mesh,
    scratch_types=[
        pltpu.VMEM((QCH, C), jnp.int32),
        pltpu.VMEM((QCH, C), jnp.int32),
        pltpu.VMEM((C, D), jnp.float32),
        pltpu.VMEM((C, D), jnp.float32),
        pltpu.VMEM((TWB, 16), jnp.float32),
        pltpu.VMEM_SHARED((ACC, D), jnp.float32),
        pltpu.SemaphoreType.DMA,
        pltpu.SemaphoreType.DMA,
        pltpu.SemaphoreType.DMA,
        pltpu.SemaphoreType.DMA,
    ],
)(_sc_hop_body)


_R = 1000


def _g_idx(i):
    return (i // 5) * (STR // _R) + (i % 5)


def _tc_layer_body(g1_ref, g2_ref, g3_ref, d_ref, x_ref, w_ref, b_ref, *out):
    dv = d_ref[0, :, :1]
    sq = jnp.where(dv > 0, 1.0 / dv, 0.0)
    h1 = g1_ref[...] * sq
    h2 = g2_ref[...] * sq
    h3 = g3_ref[...] * sq
    big = jnp.concatenate([x_ref[...], h1, h2, h3], axis=1)
    o = jnp.dot(big, w_ref[...], preferred_element_type=jnp.float32) + b_ref[...]
    xn = jnp.where(o > 0, o, 0.25 * o)
    out[0][...] = xn
    if len(out) > 1:
        out[1][...] = xn * dv


def _tc_layer(g1, g2, g3, dinv, x, wstack, b, last):
    out_shape = [jax.ShapeDtypeStruct((N, D), jnp.float32)]
    out_specs = [pl.BlockSpec((_R, D), lambda i: (i, 0))]
    if not last:
        out_shape.append(jax.ShapeDtypeStruct((GPAD, D), jnp.float32))
        out_specs.append(pl.BlockSpec((_R, D), lambda i: (_g_idx(i), 0)))
    g_spec = pl.BlockSpec((_R, D), lambda i: (_g_idx(i), 0))
    res = pl.pallas_call(
        _tc_layer_body,
        grid=(N // _R,),
        in_specs=[
            g_spec,
            g_spec,
            g_spec,
            pl.BlockSpec((1, _R, 16), lambda i: (i // 5, i % 5, 0)),
            pl.BlockSpec((_R, D), lambda i: (i, 0)),
            pl.BlockSpec((4 * D, D), lambda i: (0, 0)),
            pl.BlockSpec((1, D), lambda i: (0, 0)),
        ],
        out_specs=out_specs if not last else out_specs[0],
        out_shape=out_shape if not last else out_shape[0],
    )(g1, g2, g3, dinv, x, wstack, b)
    return res


def kernel(category, noise, edge_index, W0, b0, W1, b1, W2, b2):
    src = edge_index[0]
    dst = edge_index[1]
    pad = EPAD - E
    ar = jnp.arange(pad, dtype=jnp.int32)
    src_p = jnp.concatenate([src, ar % N]).reshape(NS, ECH, C)
    dst_p = jnp.concatenate([dst, jnp.full((pad,), N, jnp.int32)]).reshape(NS, ECH, C)

    x0 = jnp.concatenate([category, noise], axis=1)
    xpad = jnp.pad(x0, ((0, GPAD - N), (0, 0)))

    g0, dinv, srcT, dstT = _sc_prep(xpad, src_p, dst_p)
    x = x0
    for l, (W, b) in enumerate(((W0, b0), (W1, b1), (W2, b2))):
        g1 = _sc_hop(g0, srcT, dstT, dinv)
        g2 = _sc_hop(g1, srcT, dstT, dinv)
        g3 = _sc_hop(g2, srcT, dstT, dinv)
        last = l == 2
        res = _tc_layer(g1, g2, g3, dinv, x, W.reshape(4 * D, D),
                        b.reshape(1, D), last)
        if last:
            return res
        x, g0 = res


# speedup vs baseline: 6.3281x; 1.0025x over previous
import functools

import jax
import jax.numpy as jnp
import numpy as np
from jax import lax
from jax.experimental import pallas as pl
from jax.experimental.pallas import tpu as pltpu
from jax.experimental.pallas import tpu_sc as plsc

N = 10000
D = 128
E = 320000
NC, NS = 2, 16
NH = 5000
STR = 6000
ACC = 5120
TWB = ACC // NS
GPAD = 20480
C = 128
ET = 20480
ECH = ET // C
EPAD = NS * ET
QCH = 40
MAGIC = np.int32(0x5F3759DF)

_mesh = plsc.VectorSubcoreMesh(core_axis_name="c", subcore_axis_name="s")

_CHUNKS = ((0, 128), (128, 128), (256, 64))


def _fill_gbuf(gbuf, val):
    z = jnp.full((16,), val, jnp.float32)

    def body(i, carry):
        for cc in range(8):
            gbuf[i, pl.ds(cc * 16, 16)] = z
        return carry

    lax.fori_loop(0, 128, body, 0)


def _zero_gbuf(gbuf):
    _fill_gbuf(gbuf, 0.0)


def _stage_quarter(src_hbm, dst_hbm, src_v, dst_v, s, c, qq):
    base = c * NH
    pltpu.sync_copy(src_hbm.at[s, pl.ds(qq * QCH, QCH)], src_v)
    pltpu.sync_copy(dst_hbm.at[s, pl.ds(qq * QCH, QCH)], dst_v)

    def body(j, carry):
        for cc in range(8):
            sl = pl.ds(cc * 16, 16)
            v = src_v[j, sl]
            src_v[j, sl] = jnp.where(v >= NH, v + (STR - NH), v)
            w = dst_v[j, sl]
            local = w - base
            ok = (local >= 0) & (local < NH)
            dst_v[j, sl] = jnp.where(ok, local, NH + (w & 63))
        return carry

    lax.fori_loop(0, QCH, body, 0)


def _newton_rsqrt(d16):
    half = d16 * 0.5
    bits = lax.bitcast_convert_type(d16, jnp.int32)
    i = MAGIC - lax.shift_right_logical(bits, 1)
    y = lax.bitcast_convert_type(i, jnp.float32)
    for _ in range(3):
        y = y * (1.5 - half * y * y)
    return jnp.where(d16 > 0.5, y, 0.0)


def _sc_prep_body(xg_hbm, src_hbm, dst_hbm, gout, dinv_out, srcT, dstT,
                  src_v, dst_v, buf1, dv, ibuf, dacc, sem0):
    c = lax.axis_index("c")
    s = lax.axis_index("s")

    _zero_gbuf(buf1)
    pltpu.sync_copy(buf1, dacc.at[pl.ds(s * TWB, 128)])
    pltpu.sync_copy(buf1, dacc.at[pl.ds(s * TWB + 128, 128)])
    pltpu.sync_copy(buf1.at[pl.ds(0, 64)], dacc.at[pl.ds(s * TWB + 256, 64)])
    _fill_gbuf(buf1, 1.0)
    plsc.subcore_barrier()

    def douter(qq, carry):
        _stage_quarter(src_hbm, dst_hbm, src_v, dst_v, s, c, qq)
        pltpu.sync_copy(src_v, srcT.at[c, s, pl.ds(qq * QCH, QCH)])
        pltpu.sync_copy(dst_v, dstT.at[c, s, pl.ds(qq * QCH, QCH)])

        def dbody(j, inner):
            pltpu.sync_copy(buf1, dacc.at[dst_v.at[j]], add=True)
            return inner

        lax.fori_loop(0, QCH, dbody, 0)
        return carry

    lax.fori_loop(0, ECH // QCH, douter, 0)
    plsc.subcore_barrier()

    for off, cnt in _CHUNKS:
        gslice = buf1.at[pl.ds(0, cnt)] if cnt != 128 else buf1
        pltpu.sync_copy(dacc.at[pl.ds(s * TWB + off, cnt)], gslice)

        def ibody(i, carry, off=off):
            d16 = buf1[i, pl.ds(0, 16)]
            dv[off + i, pl.ds(0, 16)] = _newton_rsqrt(d16)
            return carry

        lax.fori_loop(0, cnt, ibody, 0)
    pltpu.sync_copy(dv, dinv_out.at[c, pl.ds(s * TWB, TWB)])

    for off, cnt in _CHUNKS:
        gslice = buf1.at[pl.ds(0, cnt)] if cnt != 128 else buf1
        nrow = c * NH + s * TWB + off

        def xbody(j, carry, nrow=nrow):
            ibuf[pl.ds(j * 16, 16)] = nrow + j * 16 + lax.iota(jnp.int32, 16)
            return carry

        lax.fori_loop(0, cnt // 16, xbody, 0)
        islice = ibuf.at[pl.ds(0, cnt)] if cnt != 128 else ibuf
        pltpu.async_copy(xg_hbm.at[islice], gslice, sem0).wait()

        def gbody(i, carry, off=off):
            d = dv[off + i, pl.ds(0, 16)][0]
            for cc in range(8):
                sl = pl.ds(cc * 16, 16)
                buf1[i, sl] = buf1[i, sl] * d
            return carry

        lax.fori_loop(0, cnt, gbody, 0)
        rows = c * STR + s * TWB + off
        pltpu.sync_copy(gslice, gout.at[pl.ds(rows, cnt)])


def _sc_hop_body(gtab, srcT, dstT, dinv_in, gout,
                 src_v, dst_v, buf0, buf1, dv, accum, sem0, sem1, sem2, sem3):
    c = lax.axis_index("c")
    s = lax.axis_index("s")
    pltpu.sync_copy(dinv_in.at[c, pl.ds(s * TWB, TWB)], dv)

    _zero_gbuf(buf1)
    pltpu.sync_copy(buf1, accum.at[pl.ds(s * TWB, 128)])
    pltpu.sync_copy(buf1, accum.at[pl.ds(s * TWB + 128, 128)])
    pltpu.sync_copy(buf1.at[pl.ds(0, 64)], accum.at[pl.ds(s * TWB + 256, 64)])
    plsc.subcore_barrier()

    def outer(qq, carry):
        pltpu.sync_copy(srcT.at[c, s, pl.ds(qq * QCH, QCH)], src_v)
        pltpu.sync_copy(dstT.at[c, s, pl.ds(qq * QCH, QCH)], dst_v)

        def body(it, inner):
            ch0 = it * 2
            ch1 = it * 2 + 1
            d0 = pltpu.async_copy(gtab.at[src_v.at[ch0]], buf0, sem0)
            d1 = pltpu.async_copy(gtab.at[src_v.at[ch1]], buf1, sem1)
            d0.wait()
            s0 = pltpu.async_copy(buf0, accum.at[dst_v.at[ch0]], sem2, add=True)
            d1.wait()
            s1 = pltpu.async_copy(buf1, accum.at[dst_v.at[ch1]], sem3, add=True)
            s0.wait()
            s1.wait()
            return inner

        lax.fori_loop(0, QCH // 2, body, 0)
        return carry

    lax.fori_loop(0, ECH // QCH, outer, 0)
    plsc.subcore_barrier()

    for off, cnt in _CHUNKS:
        gslice = buf1.at[pl.ds(0, cnt)] if cnt != 128 else buf1
        pltpu.sync_copy(accum.at[pl.ds(s * TWB + off, cnt)], gslice)

        def sbody(i, carry, off=off):
            d = dv[off + i, pl.ds(0, 16)][0]
            d2 = d * d
            for cc in range(8):
                sl = pl.ds(cc * 16, 16)
                buf1[i, sl] = buf1[i, sl] * d2
            return carry

        lax.fori_loop(0, cnt, sbody, 0)
        rows = c * STR + s * TWB + off
        pltpu.sync_copy(gslice, gout.at[pl.ds(rows, cnt)])


_sc_prep = functools.partial(
    pl.kernel,
    out_type=(
        jax.ShapeDtypeStruct((GPAD, D), jnp.float32),
        jax.ShapeDtypeStruct((NC, ACC, 16), jnp.float32),
        jax.ShapeDtypeStruct((NC, NS, ECH, C), jnp.int32),
        jax.ShapeDtypeStruct((NC, NS, ECH, C), jnp.int32),
    ),
    mesh=_mesh,
    scratch_types=[
        pltpu.VMEM((QCH, C), jnp.int32),
        pltpu.VMEM((QCH, C), jnp.int32),
        pltpu.VMEM((C, D), jnp.float32),
        pltpu.VMEM((TWB, 16), jnp.float32),
        pltpu.VMEM((C,), jnp.int32),
        pltpu.VMEM_SHARED((ACC, D), jnp.float32),
        pltpu.SemaphoreType.DMA,
    ],
)(lambda xg, srch, dsth, gout, dinv_out, srcT, dstT, *scr: _sc_prep_body(
    xg, srch, dsth, gout, dinv_out, srcT, dstT, *scr))

_sc_hop = functools.partial(
    pl.kernel,
    out_type=jax.ShapeDtypeStruct((GPAD, D), jnp.float32),
    mesh=_mesh,
    scratch_types=[
        pltpu.VMEM((QCH, C), jnp.int32),
        pltpu.VMEM((QCH, C), jnp.int32),
        pltpu.VMEM((C, D), jnp.float32),
        pltpu.VMEM((C, D), jnp.float32),
        pltpu.VMEM((TWB, 16), jnp.float32),
        pltpu.VMEM_SHARED((ACC, D), jnp.float32),
        pltpu.SemaphoreType.DMA,
        pltpu.SemaphoreType.DMA,
        pltpu.SemaphoreType.DMA,
        pltpu.SemaphoreType.DMA,
    ],
)(_sc_hop_body)


_R = 1000


def _g_idx(i):
    return (i // 5) * (STR // _R) + (i % 5)


def _tc_layer_body(g1_ref, g2_ref, g3_ref, d_ref, x_ref, w_ref, b_ref, *out):
    dv = d_ref[0, :, :1]
    sq = jnp.where(dv > 0, 1.0 / dv, 0.0)
    h1 = g1_ref[...] * sq
    h2 = g2_ref[...] * sq
    h3 = g3_ref[...] * sq
    big = jnp.concatenate([x_ref[...], h1, h2, h3], axis=1)
    o = jnp.dot(big, w_ref[...], preferred_element_type=jnp.float32) + b_ref[...]
    xn = jnp.where(o > 0, o, 0.25 * o)
    out[0][...] = xn
    if len(out) > 1:
        out[1][...] = xn * dv


def _tc_layer(g1, g2, g3, dinv, x, wstack, b, last):
    out_shape = [jax.ShapeDtypeStruct((N, D), jnp.float32)]
    out_specs = [pl.BlockSpec((_R, D), lambda i: (i, 0))]
    if not last:
        out_shape.append(jax.ShapeDtypeStruct((GPAD, D), jnp.float32))
        out_specs.append(pl.BlockSpec((_R, D), lambda i: (_g_idx(i), 0)))
    g_spec = pl.BlockSpec((_R, D), lambda i: (_g_idx(i), 0))
    res = pl.pallas_call(
        _tc_layer_body,
        grid=(N // _R,),
        in_specs=[
            g_spec,
            g_spec,
            g_spec,
            pl.BlockSpec((1, _R, 16), lambda i: (i // 5, i % 5, 0)),
            pl.BlockSpec((_R, D), lambda i: (i, 0)),
            pl.BlockSpec((4 * D, D), lambda i: (0, 0)),
            pl.BlockSpec((1, D), lambda i: (0, 0)),
        ],
        out_specs=out_specs if not last else out_specs[0],
        out_shape=out_shape if not last else out_shape[0],
    )(g1, g2, g3, dinv, x, wstack, b)
    return res


def kernel(category, noise, edge_index, W0, b0, W1, b1, W2, b2):
    src = edge_index[0]
    dst = edge_index[1]
    pad = EPAD - E
    ar = jnp.arange(pad, dtype=jnp.int32)
    src_p = jnp.concatenate([src, ar % N]).reshape(NS, ECH, C)
    dst_p = jnp.concatenate([dst, jnp.full((pad,), N, jnp.int32)]).reshape(NS, ECH, C)

    x0 = jnp.concatenate([category, noise], axis=1)
    xpad = jnp.pad(x0, ((0, GPAD - N), (0, 0)))

    g0, dinv, srcT, dstT = _sc_prep(xpad, src_p, dst_p)
    x = x0
    for l, (W, b) in enumerate(((W0, b0), (W1, b1), (W2, b2))):
        g1 = _sc_hop(g0, srcT, dstT, dinv)
        g2 = _sc_hop(g1, srcT, dstT, dinv)
        g3 = _sc_hop(g2, srcT, dstT, dinv)
        last = l == 2
        res = _tc_layer(g1, g2, g3, dinv, x, W.reshape(4 * D, D),
                        b.reshape(1, D), last)
        if last:
            return res
        x, g0 = res
